# baseline (speedup 1.0000x reference)
import functools

import jax
import jax.numpy as jnp
from jax import lax
from jax.experimental import pallas as pl
from jax.experimental.pallas import tpu as pltpu
from jax.experimental.pallas import tpu_sc as plsc

_VOCAB = 100000
_EMBED = 25
_BATCH = 4096
_SEQ = 200
_B = _BATCH * _SEQ
_NW = 32
_BPW = _B // _NW
_CH = 1600
_NCH = _BPW // _CH
_EPAD = 32
_UNROLL = 8

_mesh = plsc.VectorSubcoreMesh(core_axis_name="c", subcore_axis_name="s")


@functools.partial(
    pl.kernel,
    mesh=_mesh,
    out_type=jax.ShapeDtypeStruct((_B * _EMBED,), jnp.float32),
    scratch_types=[
        pltpu.VMEM((_CH,), jnp.int32),
        pltpu.VMEM((_CH, _EPAD), jnp.float32),
        pltpu.VMEM((_CH * _EMBED + 16,), jnp.float32),
        pltpu.SemaphoreType.DMA,
    ],
    compiler_params=pltpu.CompilerParams(
        use_tc_tiling_on_sc=False, needs_layout_passes=False),
)
def _gather_kernel(table_hbm, idx_hbm, out_hbm, idx_v, rows_v, cmp_v, sem):
    wid = lax.axis_index("s") * 2 + lax.axis_index("c")
    base = wid * _BPW
    tail_mask = lax.iota(jnp.int32, 16) < (_EMBED - 16)

    def body(i, carry):
        off = base + i * _CH
        pltpu.sync_copy(idx_hbm.at[pl.ds(off, _CH)], idx_v)
        pltpu.async_copy(table_hbm.at[idx_v], rows_v, sem).wait()

        def row_body(g, carry2):
            for u in range(_UNROLL):
                r = g * _UNROLL + u
                head = rows_v[r, pl.ds(0, 16)]
                tail = rows_v[r, pl.ds(16, 16)]
                cmp_v[pl.ds(r * _EMBED, 16)] = head
                plsc.store_compressed(
                    cmp_v.at[pl.ds(r * _EMBED + 16, 16)], tail,
                    mask=tail_mask)
            return carry2

        lax.fori_loop(0, _CH // _UNROLL, row_body, 0)
        pltpu.sync_copy(cmp_v.at[pl.ds(0, _CH * _EMBED)],
                        out_hbm.at[pl.ds(off * _EMBED, _CH * _EMBED)])
        return carry

    lax.fori_loop(0, _NCH, body, 0)


def kernel(x, table):
    idx = x.astype(jnp.int32).reshape(_B)
    table_pad = jnp.pad(table, ((0, 0), (0, _EPAD - _EMBED)))
    out = _gather_kernel(table_pad, idx)
    return out.reshape(_BATCH, _SEQ, _EMBED)


# speedup vs baseline: 1.3301x; 1.3301x over previous
import functools

import jax
import jax.numpy as jnp
from jax import lax
from jax.experimental import pallas as pl
from jax.experimental.pallas import tpu as pltpu
from jax.experimental.pallas import tpu_sc as plsc

_VOCAB = 100000
_EMBED = 25
_BATCH = 4096
_SEQ = 200
_B = _BATCH * _SEQ
_NW = 32
_ST = _SEQ // 8
_BT = _BATCH // 128
_NBLK = _ST * _BT
_BLKW = _NBLK // _NW
_EPAD = 32

_mesh = plsc.VectorSubcoreMesh(core_axis_name="c", subcore_axis_name="s")


@functools.partial(
    pl.kernel,
    mesh=_mesh,
    out_type=jax.ShapeDtypeStruct((_EMBED, _B), jnp.float32),
    scratch_types=[
        pltpu.VMEM((128, 8), jnp.int32),
        pltpu.VMEM((1024,), jnp.int32),
        pltpu.VMEM((1024, _EPAD), jnp.float32),
        pltpu.VMEM((_EMBED, 1024), jnp.float32),
        pltpu.SemaphoreType.DMA,
    ],
    compiler_params=pltpu.CompilerParams(
        use_tc_tiling_on_sc=False, needs_layout_passes=False),
)
def _gather_kernel(table_hbm, x_hbm, out_hbm, idx2_v, idx1_v, rows_v,
                   out_v, sem):
    wid = lax.axis_index("s") * 2 + lax.axis_index("c")
    iota = lax.iota(jnp.int32, 16)

    def blk_body(t, carry):
        blk = wid * _BLKW + t
        st = blk // _BT
        bt = blk % _BT
        pltpu.sync_copy(
            x_hbm.at[pl.ds(bt * 128, 128), pl.ds(st * 8, 8)], idx2_v)

        def flat_body(m, c):
            w = m * 16 + iota
            v = plsc.load_gather(idx2_v, [w >> 3, w & 7])
            idx1_v[pl.ds(m * 16, 16)] = v
            return c

        lax.fori_loop(0, 64, flat_body, 0)
        pltpu.async_copy(table_hbm.at[idx1_v], rows_v, sem).wait()

        def tr_body(m, c):
            si = m // 8
            u = m % 8
            base = si * 128 + u * 16
            q = iota * 8 + (u * 128 + si)
            for e in range(_EMBED):
                ev = jnp.full((16,), e, jnp.int32)
                v = plsc.load_gather(rows_v, [q, ev])
                out_v[e, pl.ds(base, 16)] = v
            return c

        lax.fori_loop(0, 64, tr_body, 0)
        pltpu.sync_copy(out_v, out_hbm.at[:, pl.ds(blk * 1024, 1024)])
        return carry

    lax.fori_loop(0, _BLKW, blk_body, 0)


def kernel(x, table):
    table_pad = jnp.pad(table, ((0, 0), (0, _EPAD - _EMBED)))
    out2 = _gather_kernel(table_pad, x.astype(jnp.int32))
    out5 = out2.reshape(_EMBED, _ST, _BT, 8, 128)
    return out5.transpose(2, 4, 1, 3, 0).reshape(_BATCH, _SEQ, _EMBED)


# speedup vs baseline: 1.8858x; 1.4178x over previous
import functools

import jax
import jax.numpy as jnp
from jax import lax
from jax.experimental import pallas as pl
from jax.experimental.pallas import tpu as pltpu
from jax.experimental.pallas import tpu_sc as plsc

_VOCAB = 100000
_EMBED = 25
_BATCH = 4096
_SEQ = 200
_B = _BATCH * _SEQ
_NW = 32
_ST = _SEQ // 8
_BT = _BATCH // 128
_NBLK = _ST * _BT
_BLKW = _NBLK // _NW
_EPAD = 32

_mesh = plsc.VectorSubcoreMesh(core_axis_name="c", subcore_axis_name="s")


@functools.partial(
    pl.kernel,
    mesh=_mesh,
    out_type=jax.ShapeDtypeStruct((_EMBED, _B), jnp.float32),
    scratch_types=[
        pltpu.VMEM((128, 8), jnp.int32),
        pltpu.VMEM((1024,), jnp.int32),
        pltpu.VMEM((1024, _EPAD), jnp.float32),
        pltpu.VMEM((_EMBED, 1024), jnp.float32),
        pltpu.SemaphoreType.DMA,
    ],
    compiler_params=pltpu.CompilerParams(
        use_tc_tiling_on_sc=False, needs_layout_passes=False),
)
def _gather_kernel(table_hbm, x_hbm, out_hbm, idx2_v, idx1_v, rows_v,
                   out_v, sem):
    wid = lax.axis_index("s") * 2 + lax.axis_index("c")
    iota = lax.iota(jnp.int32, 16)

    def blk_body(t, carry):
        blk = wid * _BLKW + t
        st = blk // _BT
        bt = blk % _BT
        pltpu.sync_copy(
            x_hbm.at[pl.ds(bt * 128, 128), pl.ds(st * 8, 8)], idx2_v)

        @plsc.parallel_loop(0, 64, unroll=4)
        def flat_body(m):
            w = m * 16 + iota
            v = plsc.load_gather(idx2_v, [w >> 3, w & 7])
            idx1_v[pl.ds(m * 16, 16)] = v

        pltpu.async_copy(table_hbm.at[idx1_v], rows_v, sem).wait()

        @plsc.parallel_loop(0, 64, unroll=2)
        def tr_body(m):
            si = m // 8
            u = m % 8
            base = si * 128 + u * 16
            q = iota * 8 + (u * 128 + si)
            for e in range(_EMBED):
                ev = jnp.full((16,), e, jnp.int32)
                v = plsc.load_gather(rows_v, [q, ev])
                out_v[e, pl.ds(base, 16)] = v
        pltpu.sync_copy(out_v, out_hbm.at[:, pl.ds(blk * 1024, 1024)])
        return carry

    lax.fori_loop(0, _BLKW, blk_body, 0)


def kernel(x, table):
    table_pad = jnp.pad(table, ((0, 0), (0, _EPAD - _EMBED)))
    out2 = _gather_kernel(table_pad, x.astype(jnp.int32))
    out5 = out2.reshape(_EMBED, _ST, _BT, 8, 128)
    return out5.transpose(2, 4, 1, 3, 0).reshape(_BATCH, _SEQ, _EMBED)


# speedup vs baseline: 3.4797x; 1.8452x over previous
import functools

import jax
import jax.numpy as jnp
from jax import lax
from jax.experimental import pallas as pl
from jax.experimental.pallas import tpu as pltpu
from jax.experimental.pallas import tpu_sc as plsc

_VOCAB = 100000
_EMBED = 25
_BATCH = 4096
_SEQ = 200
_B = _BATCH * _SEQ
_NW = 32
_ST = _SEQ // 8
_BT = _BATCH // 128
_CH = 8192
_NCH = _B // _CH

_mesh = plsc.VectorSubcoreMesh(core_axis_name="c", subcore_axis_name="s")


@functools.partial(
    pl.kernel,
    mesh=_mesh,
    out_type=jax.ShapeDtypeStruct((_EMBED, _B), jnp.float32),
    scratch_types=[
        pltpu.VMEM((_VOCAB,), jnp.float32),
        pltpu.VMEM((_CH,), jnp.int32),
        pltpu.VMEM((_CH,), jnp.float32),
    ],
    compiler_params=pltpu.CompilerParams(
        use_tc_tiling_on_sc=False, needs_layout_passes=False),
)
def _gather_kernel(tblt_hbm, idx_hbm, out_hbm, tbl_v, idx_v, out_v):
    wid = lax.axis_index("s") * 2 + lax.axis_index("c")
    iota = lax.iota(jnp.int32, 16)

    @pl.when(wid < _EMBED)
    def _():
        pltpu.sync_copy(tblt_hbm.at[wid], tbl_v)

        def ch_body(c, carry):
            off = c * _CH
            pltpu.sync_copy(idx_hbm.at[pl.ds(off, _CH)], idx_v)

            @plsc.parallel_loop(0, _CH // 16, unroll=8)
            def gat_body(m):
                ids = idx_v[pl.ds(m * 16, 16)]
                out_v[pl.ds(m * 16, 16)] = plsc.load_gather(tbl_v, [ids])

            pltpu.sync_copy(out_v, out_hbm.at[wid, pl.ds(off, _CH)])
            return carry

        lax.fori_loop(0, _NCH, ch_body, 0)


def kernel(x, table):
    xp = (x.astype(jnp.int32)
          .reshape(_BT, 128, _ST, 8)
          .transpose(2, 0, 3, 1)
          .reshape(_B))
    table_t = table.T
    out2 = _gather_kernel(table_t, xp)
    out5 = out2.reshape(_EMBED, _ST, _BT, 8, 128)
    return out5.transpose(2, 4, 1, 3, 0).reshape(_BATCH, _SEQ, _EMBED)


# speedup vs baseline: 7.6343x; 2.1939x over previous
import functools

import jax
import jax.numpy as jnp
from jax import lax
from jax.experimental import pallas as pl
from jax.experimental.pallas import tpu as pltpu
from jax.experimental.pallas import tpu_sc as plsc

_VOCAB = 100000
_EMBED = 25
_BATCH = 4096
_SEQ = 200
_B = _BATCH * _SEQ
_NW = 32
_ST = _SEQ // 8
_BT = _BATCH // 128
_CH = 4096
_NCH = _B // _CH
_HALF = _B // 4
_NCHH = _HALF // _CH
_SLAB = _HALF // 16

_mesh = plsc.VectorSubcoreMesh(core_axis_name="c", subcore_axis_name="s")


@functools.partial(
    pl.kernel,
    mesh=_mesh,
    out_type=jax.ShapeDtypeStruct((_EMBED, _B), jnp.float32),
    scratch_types=[
        pltpu.VMEM_SHARED((_HALF,), jnp.int32),
        pltpu.VMEM((_VOCAB,), jnp.float32),
        pltpu.VMEM((2, _CH), jnp.int32),
        pltpu.VMEM((2, _CH), jnp.float32),
        pltpu.SemaphoreType.DMA,
        pltpu.SemaphoreType.DMA,
        pltpu.SemaphoreType.DMA,
        pltpu.SemaphoreType.DMA,
    ],
    compiler_params=pltpu.CompilerParams(
        use_tc_tiling_on_sc=False, needs_layout_passes=False),
)
def _gather_kernel(tblt_hbm, idx_hbm, out_hbm, idx_sh, tbl_v, idx_v, out_v,
                   sem_i0, sem_i1, sem_o0, sem_o1):
    wid = lax.axis_index("s") * 2 + lax.axis_index("c")
    sid = lax.axis_index("s")
    sem_i = (sem_i0, sem_i1)
    sem_o = (sem_o0, sem_o1)

    @pl.when(wid < _EMBED)
    def _():
        pltpu.sync_copy(tblt_hbm.at[wid], tbl_v)

    for h in range(4):
        pltpu.sync_copy(
            idx_hbm.at[pl.ds(h * _HALF + sid * _SLAB, _SLAB)],
            idx_sh.at[pl.ds(sid * _SLAB, _SLAB)])
        plsc.subcore_barrier()

        @pl.when(wid < _EMBED)
        def _():
            def idx_copy(k, p):
                return pltpu.make_async_copy(
                    idx_sh.at[pl.ds(k * _CH, _CH)], idx_v.at[p], sem_i[p])

            def out_copy(k, p):
                return pltpu.make_async_copy(
                    out_v.at[p],
                    out_hbm.at[wid, pl.ds(h * _HALF + k * _CH, _CH)],
                    sem_o[p])

            idx_copy(0, 0).start()
            idx_copy(1, 1).start()

            def ch_body(t, carry):
                for p in range(2):
                    k = t * 2 + p
                    idx_copy(k, p).wait()

                    @pl.when(k >= 2)
                    def _():
                        out_copy(k - 2, p).wait()

                    @plsc.parallel_loop(0, _CH // 16, unroll=8)
                    def gat_body(m):
                        ids = idx_v[p, pl.ds(m * 16, 16)]
                        out_v[p, pl.ds(m * 16, 16)] = plsc.load_gather(
                            tbl_v, [ids])

                    out_copy(k, p).start()

                    @pl.when(k + 2 < _NCHH)
                    def _():
                        idx_copy(k + 2, p).start()

                return carry

            lax.fori_loop(0, _NCHH // 2, ch_body, 0)
            out_copy(_NCHH - 2, 0).wait()
            out_copy(_NCHH - 1, 1).wait()

        plsc.subcore_barrier()


def kernel(x, table):
    xp = (x.astype(jnp.int32)
          .reshape(_BT, 128, _ST, 8)
          .transpose(2, 0, 3, 1)
          .reshape(_B))
    table_t = table.T
    out2 = _gather_kernel(table_t, xp)
    out5 = out2.reshape(_EMBED, _ST, _BT, 8, 128)
    return out5.transpose(2, 4, 1, 3, 0).reshape(_BATCH, _SEQ, _EMBED)


# speedup vs baseline: 7.7159x; 1.0107x over previous
import functools

import jax
import jax.numpy as jnp
from jax import lax
from jax.experimental import pallas as pl
from jax.experimental.pallas import tpu as pltpu
from jax.experimental.pallas import tpu_sc as plsc

_VOCAB = 100000
_EMBED = 25
_BATCH = 4096
_SEQ = 200
_B = _BATCH * _SEQ
_NW = 32
_ST = _SEQ // 8
_BT = _BATCH // 128
_CH = 4096
_NCH = _B // _CH
_HALF = _B // 4
_NCHH = _HALF // _CH
_SLAB = _HALF // 16

_mesh = plsc.VectorSubcoreMesh(core_axis_name="c", subcore_axis_name="s")


@functools.partial(
    pl.kernel,
    mesh=_mesh,
    out_type=jax.ShapeDtypeStruct((_EMBED, _B), jnp.float32),
    scratch_types=[
        pltpu.VMEM_SHARED((_HALF,), jnp.int32),
        pltpu.VMEM((_VOCAB,), jnp.float32),
        pltpu.VMEM((2, _CH), jnp.int32),
        pltpu.VMEM((2, _CH), jnp.float32),
        pltpu.SemaphoreType.DMA,
        pltpu.SemaphoreType.DMA,
        pltpu.SemaphoreType.DMA,
        pltpu.SemaphoreType.DMA,
    ],
    compiler_params=pltpu.CompilerParams(
        use_tc_tiling_on_sc=False, needs_layout_passes=False),
)
def _gather_kernel(tblt_hbm, idx_hbm, out_hbm, idx_sh, tbl_v, idx_v, out_v,
                   sem_i0, sem_i1, sem_o0, sem_o1):
    wid = lax.axis_index("s") * 2 + lax.axis_index("c")
    sid = lax.axis_index("s")
    sem_i = (sem_i0, sem_i1)
    sem_o = (sem_o0, sem_o1)

    @pl.when(wid < _EMBED)
    def _():
        pltpu.make_async_copy(tblt_hbm.at[wid], tbl_v, sem_i0).start()

    for h in range(4):
        pltpu.sync_copy(
            idx_hbm.at[pl.ds(h * _HALF + sid * _SLAB, _SLAB)],
            idx_sh.at[pl.ds(sid * _SLAB, _SLAB)])
        if h == 0:
            @pl.when(wid < _EMBED)
            def _():
                pltpu.make_async_copy(tblt_hbm.at[wid], tbl_v, sem_i0).wait()
        plsc.subcore_barrier()

        @pl.when(wid < _EMBED)
        def _():
            def idx_copy(k, p):
                return pltpu.make_async_copy(
                    idx_sh.at[pl.ds(k * _CH, _CH)], idx_v.at[p], sem_i[p])

            def out_copy(k, p):
                return pltpu.make_async_copy(
                    out_v.at[p],
                    out_hbm.at[wid, pl.ds(h * _HALF + k * _CH, _CH)],
                    sem_o[p])

            idx_copy(0, 0).start()
            idx_copy(1, 1).start()

            def ch_body(t, carry):
                for p in range(2):
                    k = t * 2 + p
                    idx_copy(k, p).wait()

                    @pl.when(k >= 2)
                    def _():
                        out_copy(k - 2, p).wait()

                    @plsc.parallel_loop(0, _CH // 16, unroll=16)
                    def gat_body(m):
                        ids = idx_v[p, pl.ds(m * 16, 16)]
                        out_v[p, pl.ds(m * 16, 16)] = plsc.load_gather(
                            tbl_v, [ids])

                    out_copy(k, p).start()

                    @pl.when(k + 2 < _NCHH)
                    def _():
                        idx_copy(k + 2, p).start()

                return carry

            lax.fori_loop(0, _NCHH // 2, ch_body, 0)
            out_copy(_NCHH - 2, 0).wait()
            out_copy(_NCHH - 1, 1).wait()

        plsc.subcore_barrier()


def kernel(x, table):
    xp = (x.astype(jnp.int32)
          .reshape(_BT, 128, _ST, 8)
          .transpose(2, 0, 3, 1)
          .reshape(_B))
    table_t = table.T
    out2 = _gather_kernel(table_t, xp)
    out5 = out2.reshape(_EMBED, _ST, _BT, 8, 128)
    return out5.transpose(2, 4, 1, 3, 0).reshape(_BATCH, _SEQ, _EMBED)


# speedup vs baseline: 7.7389x; 1.0030x over previous
import functools

import jax
import jax.numpy as jnp
from jax import lax
from jax.experimental import pallas as pl
from jax.experimental.pallas import tpu as pltpu
from jax.experimental.pallas import tpu_sc as plsc

_VOCAB = 100000
_EMBED = 25
_BATCH = 4096
_SEQ = 200
_B = _BATCH * _SEQ
_ST = _SEQ // 8
_BT = _BATCH // 128
_CH = 4096
_HALF = _B // 4
_NCHH = _HALF // _CH
_SLAB = _HALF // 16

_mesh = plsc.VectorSubcoreMesh(core_axis_name="c", subcore_axis_name="s")


@functools.partial(
    pl.kernel,
    mesh=_mesh,
    out_type=jax.ShapeDtypeStruct((_EMBED, _B), jnp.float32),
    scratch_types=[
        pltpu.VMEM_SHARED((_HALF,), jnp.int32),
        pltpu.VMEM((_VOCAB,), jnp.float32),
        pltpu.VMEM((2, _CH), jnp.int32),
        pltpu.VMEM((2, _CH), jnp.float32),
        pltpu.SemaphoreType.DMA,
        pltpu.SemaphoreType.DMA,
        pltpu.SemaphoreType.DMA,
        pltpu.SemaphoreType.DMA,
    ],
    compiler_params=pltpu.CompilerParams(
        use_tc_tiling_on_sc=False, needs_layout_passes=False),
)
def _gather_kernel(tblt_hbm, idx_hbm, out_hbm, idx_sh, tbl_v, idx_v, out_v,
                   sem_i0, sem_i1, sem_o0, sem_o1):
    wid = lax.axis_index("s") * 2 + lax.axis_index("c")
    sid = lax.axis_index("s")
    sem_i = (sem_i0, sem_i1)
    sem_o = (sem_o0, sem_o1)

    @pl.when(wid < _EMBED)
    def _():
        pltpu.make_async_copy(tblt_hbm.at[wid], tbl_v, sem_i0).start()

    for h in range(4):
        pltpu.sync_copy(
            idx_hbm.at[pl.ds(h * _HALF + sid * _SLAB, _SLAB)],
            idx_sh.at[pl.ds(sid * _SLAB, _SLAB)])
        if h == 0:
            @pl.when(wid < _EMBED)
            def _():
                pltpu.make_async_copy(tblt_hbm.at[wid], tbl_v, sem_i0).wait()
        plsc.subcore_barrier()

        @pl.when(wid < _EMBED)
        def _():
            def idx_copy(k, p):
                return pltpu.make_async_copy(
                    idx_sh.at[pl.ds(k * _CH, _CH)], idx_v.at[p], sem_i[p])

            def out_copy(k, p):
                return pltpu.make_async_copy(
                    out_v.at[p],
                    out_hbm.at[wid, pl.ds(h * _HALF + k * _CH, _CH)],
                    sem_o[p])

            idx_copy(0, 0).start()
            idx_copy(1, 1).start()

            def ch_body(t, carry):
                for p in range(2):
                    k = t * 2 + p
                    idx_copy(k, p).wait()

                    @pl.when(k >= 2)
                    def _():
                        out_copy(k - 2, p).wait()

                    @plsc.parallel_loop(0, _CH // 16, unroll=16)
                    def gat_body(m):
                        ids = idx_v[p, pl.ds(m * 16, 16)]
                        out_v[p, pl.ds(m * 16, 16)] = plsc.load_gather(
                            tbl_v, [ids])

                    out_copy(k, p).start()

                    @pl.when(k + 2 < _NCHH)
                    def _():
                        idx_copy(k + 2, p).start()

                return carry

            lax.fori_loop(0, _NCHH // 2, ch_body, 0)
            out_copy(_NCHH - 2, 0).wait()
            out_copy(_NCHH - 1, 1).wait()

        plsc.subcore_barrier()


def kernel(x, table):
    xp = (x.astype(jnp.int32)
          .reshape(_BT, 128, _ST, 8)
          .transpose(2, 0, 3, 1)
          .reshape(_B))
    table_t = table.T
    out2 = _gather_kernel(table_t, xp)
    out5 = out2.reshape(_EMBED, _ST, _BT, 8, 128)
    return out5.transpose(2, 4, 1, 3, 0).reshape(_BATCH, _SEQ, _EMBED)
